# baseline (speedup 1.0000x reference)
import functools

import jax
import jax.numpy as jnp
from jax.experimental import pallas as pl
from jax.experimental.pallas import tpu as pltpu

_KSIZE = 5
_EPS = 1e-6


def _shift(a, s):
    return jnp.concatenate([a[:, s:], a[:, :s]], axis=1)


def _body(x_ref, wxt_ref, wgt_ref, o_ref, stack_ref, xbf_ref, *, h, w):
    hw = h * w
    r = _KSIZE // 2
    c = x_ref.shape[2]
    kk = _KSIZE * _KSIZE
    nhalf = kk // 2 + 1

    iota = jax.lax.broadcasted_iota(jnp.int32, (1, hw), 1)
    colv = iota & (w - 1)

    rowm = {}
    colm = {}
    for d in range(-r, r + 1):
        rv = (iota + d * w >= 0) & (iota + d * w < hw)
        cv = (colv + d >= 0) & (colv + d < w)
        rowm[d] = jnp.where(rv, 1.0, 0.0).astype(jnp.float32)
        colm[d] = jnp.where(cv, 1.0, 0.0).astype(jnp.float32)

    taps = []
    for t in range(nhalf):
        di = t // _KSIZE - r
        dj = t % _KSIZE - r
        taps.append((di, dj, di * w + dj))

    cc = 256

    xbf_ref[...] = x_ref[0].astype(jnp.bfloat16)

    def cbody(ci, raws):
        xcb = xbf_ref[:, pl.ds(ci * cc, cc)].T
        xc32 = pltpu.bitcast(xcb, jnp.int32)
        out = []
        for t, (_, _, s) in enumerate(taps):
            if s != 0:
                xs = pltpu.bitcast(_shift(xc32, s), jnp.bfloat16)
            else:
                xs = xcb
            p = jnp.sum(xcb * xs, axis=0, keepdims=True,
                        dtype=jnp.bfloat16)
            out.append(raws[t] + p.astype(jnp.float32))
        return tuple(out)

    zero_row = jnp.zeros((1, hw), jnp.float32)
    raws = jax.lax.fori_loop(0, c // cc, cbody, (zero_row,) * nhalf)

    gs = [None] * kk
    for t, (di, dj, _) in enumerate(taps):
        gs[t] = raws[t] * rowm[di] * colm[dj]
    for t in range(nhalf, kk):
        di = t // _KSIZE - r
        dj = t % _KSIZE - r
        s = di * w + dj
        gs[t] = _shift(gs[kk - 1 - t], s) * rowm[di] * colm[dj]

    ssq = gs[0] * gs[0]
    for t in range(1, kk):
        ssq = ssq + gs[t] * gs[t]
    rn = jax.lax.rsqrt(ssq + _EPS)

    rows = [g * rn for g in gs]
    rows.append(jnp.ones((1, hw), jnp.float32))
    stack_ref[0:kk + 1, :] = jnp.concatenate(rows, axis=0).astype(jnp.bfloat16)

    nw = 1024
    for m0 in range(0, hw, nw):
        acc = jnp.dot(xbf_ref[m0:m0 + nw, :], wxt_ref[...],
                      preferred_element_type=jnp.float32)
        acc = acc + jax.lax.dot_general(
            stack_ref[0:kk + 1, m0:m0 + nw], wgt_ref[...],
            ((((0,), (0,))), ((), ())),
            preferred_element_type=jnp.float32)
        o_ref[0, m0:m0 + nw, :] = jnp.maximum(acc, 0.0)


def kernel(x, W, bias):
    b, c, h, w = x.shape
    hw = h * w
    hidden = W.shape[0]
    kk = _KSIZE * _KSIZE

    x_pm = jnp.transpose(x, (0, 2, 3, 1)).reshape(b, hw, c)
    wxt = W[:, :c].T.astype(jnp.bfloat16)
    wgt = jnp.concatenate([W[:, c:], bias[:, None]],
                          axis=1).T.astype(jnp.bfloat16)

    out_pm = pl.pallas_call(
        functools.partial(_body, h=h, w=w),
        grid=(b,),
        in_specs=[
            pl.BlockSpec((1, hw, c), lambda i: (i, 0, 0)),
            pl.BlockSpec((c, hidden), lambda i: (0, 0)),
            pl.BlockSpec((kk + 1, hidden), lambda i: (0, 0)),
        ],
        out_specs=pl.BlockSpec((1, hw, hidden), lambda i: (i, 0, 0)),
        out_shape=jax.ShapeDtypeStruct((b, hw, hidden), jnp.float32),
        scratch_shapes=[pltpu.VMEM((32, hw), jnp.bfloat16),
                        pltpu.VMEM((hw, c), jnp.bfloat16)],
        compiler_params=pltpu.CompilerParams(
            dimension_semantics=("parallel",),
            vmem_limit_bytes=100 * 1024 * 1024,
        ),
    )(x_pm, wxt, wgt)
    return out_pm.reshape(b, h, w, hidden).transpose(0, 3, 1, 2)


# speedup vs baseline: 1.1107x; 1.1107x over previous
import functools

import jax
import jax.numpy as jnp
from jax.experimental import pallas as pl
from jax.experimental.pallas import tpu as pltpu

_KSIZE = 5
_EPS = 1e-6


def _shift(a, s):
    return jnp.concatenate([a[:, s:], a[:, :s]], axis=1)


def _body(x_ref, wxt_ref, wgt_ref, o_ref, stack_ref, *, h, w):
    hw = h * w
    r = _KSIZE // 2
    c = x_ref.shape[2]
    kk = _KSIZE * _KSIZE
    nhalf = kk // 2 + 1

    iota = jax.lax.broadcasted_iota(jnp.int32, (1, hw), 1)
    colv = iota & (w - 1)

    rowm = {}
    colm = {}
    for d in range(-r, r + 1):
        rv = (iota + d * w >= 0) & (iota + d * w < hw)
        cv = (colv + d >= 0) & (colv + d < w)
        rowm[d] = jnp.where(rv, 1.0, 0.0).astype(jnp.float32)
        colm[d] = jnp.where(cv, 1.0, 0.0).astype(jnp.float32)

    taps = []
    for t in range(nhalf):
        di = t // _KSIZE - r
        dj = t % _KSIZE - r
        taps.append((di, dj, di * w + dj))

    cc = 512

    raws = [None] * nhalf
    for c0 in range(0, c, cc):
        xcb = x_ref[0, :, c0:c0 + cc].astype(jnp.bfloat16).T
        xc32 = pltpu.bitcast(xcb, jnp.int32)
        for t, (_, _, s) in enumerate(taps):
            if s != 0:
                xs = pltpu.bitcast(_shift(xc32, s), jnp.bfloat16)
            else:
                xs = xcb
            p = jnp.sum(xcb * xs, axis=0, keepdims=True,
                        dtype=jnp.bfloat16)
            pf = p.astype(jnp.float32)
            raws[t] = pf if c0 == 0 else raws[t] + pf

    gs = [None] * kk
    for t, (di, dj, _) in enumerate(taps):
        gs[t] = raws[t] * rowm[di] * colm[dj]
    for t in range(nhalf, kk):
        di = t // _KSIZE - r
        dj = t % _KSIZE - r
        s = di * w + dj
        gs[t] = _shift(gs[kk - 1 - t], s) * rowm[di] * colm[dj]

    ssq = gs[0] * gs[0]
    for t in range(1, kk):
        ssq = ssq + gs[t] * gs[t]
    rn = jax.lax.rsqrt(ssq + _EPS)

    rows = [g * rn for g in gs]
    rows.append(jnp.ones((1, hw), jnp.float32))
    stack_ref[0:kk + 1, :] = jnp.concatenate(rows, axis=0).astype(jnp.bfloat16)

    nw = 1024
    for m0 in range(0, hw, nw):
        acc = jnp.dot(x_ref[0, m0:m0 + nw, :].astype(jnp.bfloat16),
                      wxt_ref[...],
                      preferred_element_type=jnp.float32)
        acc = acc + jax.lax.dot_general(
            stack_ref[0:kk + 1, m0:m0 + nw], wgt_ref[...],
            ((((0,), (0,))), ((), ())),
            preferred_element_type=jnp.float32)
        o_ref[0, m0:m0 + nw, :] = jnp.maximum(acc, 0.0)


def kernel(x, W, bias):
    b, c, h, w = x.shape
    hw = h * w
    hidden = W.shape[0]
    kk = _KSIZE * _KSIZE

    x_pm = jnp.transpose(x, (0, 2, 3, 1)).reshape(b, hw, c)
    wxt = W[:, :c].T.astype(jnp.bfloat16)
    wgt = jnp.concatenate([W[:, c:], bias[:, None]],
                          axis=1).T.astype(jnp.bfloat16)

    out_pm = pl.pallas_call(
        functools.partial(_body, h=h, w=w),
        grid=(b,),
        in_specs=[
            pl.BlockSpec((1, hw, c), lambda i: (i, 0, 0)),
            pl.BlockSpec((c, hidden), lambda i: (0, 0)),
            pl.BlockSpec((kk + 1, hidden), lambda i: (0, 0)),
        ],
        out_specs=pl.BlockSpec((1, hw, hidden), lambda i: (i, 0, 0)),
        out_shape=jax.ShapeDtypeStruct((b, hw, hidden), jnp.float32),
        scratch_shapes=[pltpu.VMEM((32, hw), jnp.bfloat16)],
        compiler_params=pltpu.CompilerParams(
            dimension_semantics=("parallel",),
            vmem_limit_bytes=100 * 1024 * 1024,
        ),
    )(x_pm, wxt, wgt)
    return out_pm.reshape(b, h, w, hidden).transpose(0, 3, 1, 2)
